# Initial kernel scaffold; baseline (speedup 1.0000x reference)
import jax
import jax.numpy as jnp
from jax.experimental import pallas as pl


def kernel(pad_kmers_id_seq, enc_graph, dec_graph, drug_graph, dis_graph, drug_sim_feat, disease_sim_feat, Wp, bp, Wd, bd, Wg1_drug, Wg2_drug, Wg1_dis, Wg2_dis, Watt, batt, qatt, Wdec1, bdec1, Wdec2, bdec2):
    raise NotImplementedError("write your pallas kernel here")


# speedup vs baseline: 3.2350x; 3.2350x over previous
import functools

import jax
import jax.numpy as jnp
from jax import lax
from jax.experimental import pallas as pl
from jax.experimental.pallas import tpu as pltpu
from jax.experimental.pallas import tpu_sc as plsc

N_NODE = 10000
E = 320000
FDIM = 128
NHID2 = 64

NC = 2
NS = 16
NW = NC * NS
CW = 128
NCH = (E + NW * CW - 1) // (NW * CW)
EPAD = NW * NCH * CW
NPAD = 10240
RPT = NPAD // NS

_mesh = plsc.VectorSubcoreMesh(core_axis_name="c", subcore_axis_name="s")
_sc_params = pltpu.CompilerParams(use_tc_tiling_on_sc=False)


def _mm_body(x_ref, w_ref, o_ref):
    o_ref[...] = jnp.dot(x_ref[...], w_ref[...], preferred_element_type=jnp.float32)


def _matmul(x, w):
    return pl.pallas_call(
        _mm_body,
        out_shape=jax.ShapeDtypeStruct((x.shape[0], w.shape[1]), jnp.float32),
    )(x, w)


def _comb1_body(p_ref, d_ref, w_ref, y2_ref, dinv_ref):
    s = p_ref[:NPAD] + p_ref[NPAD:]
    deg = d_ref[:NPAD] + d_ref[NPAD:]
    dinv = 1.0 / jnp.maximum(deg, 1.0)
    h = jnp.maximum(s * dinv[:, 0:1], 0.0)
    y2_ref[...] = jnp.dot(h, w_ref[...], preferred_element_type=jnp.float32)
    dinv_ref[...] = dinv


def _combine1(p, d, w):
    return pl.pallas_call(
        _comb1_body,
        out_shape=(
            jax.ShapeDtypeStruct((NPAD, NHID2), jnp.float32),
            jax.ShapeDtypeStruct((NPAD, 16), jnp.float32),
        ),
    )(p, d, w)


def _comb2_body(q_ref, dinv_ref, w_ref, b_ref, sim_ref, a_ref):
    sim = (q_ref[:NPAD] + q_ref[NPAD:]) * dinv_ref[:, 0:1]
    sim_ref[...] = sim
    a_ref[...] = jnp.dot(sim, w_ref[...], preferred_element_type=jnp.float32) + b_ref[...]


def _combine2(q, dinv, w, b2d):
    return pl.pallas_call(
        _comb2_body,
        out_shape=(
            jax.ShapeDtypeStruct((NPAD, NHID2), jnp.float32),
            jax.ShapeDtypeStruct((NPAD, NHID2), jnp.float32),
        ),
    )(q, dinv, w, b2d)


_DEC_BLK = 4096


def _dec_body(e_ref, w_ref, b_ref, o_ref):
    h = jnp.maximum(e_ref[...], 0.0)
    o_ref[...] = jnp.dot(h, w_ref[...], preferred_element_type=jnp.float32) + b_ref[...]


def _decoder_mm(ef, w, b2d):
    grid = EPAD // _DEC_BLK
    return pl.pallas_call(
        _dec_body,
        grid=(grid,),
        in_specs=[
            pl.BlockSpec((_DEC_BLK, NHID2), lambda i: (i, 0)),
            pl.BlockSpec((NHID2, 2), lambda i: (0, 0)),
            pl.BlockSpec((1, 2), lambda i: (0, 0)),
        ],
        out_specs=pl.BlockSpec((_DEC_BLK, 2), lambda i: (i, 0)),
        out_shape=jax.ShapeDtypeStruct((EPAD, 2), jnp.float32),
    )(ef, w, b2d)


def _segsum_deg_factory(D):
    @functools.partial(
        pl.kernel,
        mesh=_mesh,
        out_type=(
            jax.ShapeDtypeStruct((NC * NPAD, D), jnp.float32),
            jax.ShapeDtypeStruct((NC * NPAD, 16), jnp.float32),
        ),
        scratch_types=[
            pltpu.VMEM((CW,), jnp.int32),
            pltpu.VMEM((CW,), jnp.int32),
            pltpu.VMEM((CW, D), jnp.float32),
            pltpu.VMEM((CW, 16), jnp.float32),
            pltpu.VMEM_SHARED((NPAD, D), jnp.float32),
            pltpu.VMEM_SHARED((NPAD, 16), jnp.float32),
            pltpu.SemaphoreType.DMA,
        ],
        compiler_params=_sc_params,
    )
    def k(y_hbm, sidx_hbm, didx_hbm, z_hbm, zd_hbm, ones_hbm,
          out_hbm, outd_hbm,
          sidx_v, didx_v, rows_v, ones_v, acc, accd, sem):
        c = lax.axis_index("c")
        s = lax.axis_index("s")
        wid = s * NC + c
        pltpu.sync_copy(z_hbm, acc.at[pl.ds(s * RPT, RPT)])
        pltpu.sync_copy(zd_hbm, accd.at[pl.ds(s * RPT, RPT)])
        pltpu.sync_copy(ones_hbm, ones_v)
        plsc.subcore_barrier()

        def body(t, carry):
            pltpu.sync_copy(sidx_hbm.at[wid, t], sidx_v)
            pltpu.async_copy(y_hbm.at[sidx_v], rows_v, sem).wait()
            pltpu.sync_copy(didx_hbm.at[wid, t], didx_v)
            pltpu.sync_copy(rows_v, acc.at[didx_v], add=True)
            pltpu.sync_copy(ones_v, accd.at[didx_v], add=True)
            return carry

        lax.fori_loop(0, NCH, body, 0)
        plsc.subcore_barrier()
        base = c * NPAD + s * RPT
        pltpu.sync_copy(acc.at[pl.ds(s * RPT, RPT)], out_hbm.at[pl.ds(base, RPT)])
        pltpu.sync_copy(accd.at[pl.ds(s * RPT, RPT)], outd_hbm.at[pl.ds(base, RPT)])

    return k


def _segsum_factory(D):
    @functools.partial(
        pl.kernel,
        mesh=_mesh,
        out_type=jax.ShapeDtypeStruct((NC * NPAD, D), jnp.float32),
        scratch_types=[
            pltpu.VMEM((CW,), jnp.int32),
            pltpu.VMEM((CW,), jnp.int32),
            pltpu.VMEM((CW, D), jnp.float32),
            pltpu.VMEM_SHARED((NPAD, D), jnp.float32),
            pltpu.SemaphoreType.DMA,
        ],
        compiler_params=_sc_params,
    )
    def k(y_hbm, sidx_hbm, didx_hbm, z_hbm, out_hbm,
          sidx_v, didx_v, rows_v, acc, sem):
        c = lax.axis_index("c")
        s = lax.axis_index("s")
        wid = s * NC + c
        pltpu.sync_copy(z_hbm, acc.at[pl.ds(s * RPT, RPT)])
        plsc.subcore_barrier()

        def body(t, carry):
            pltpu.sync_copy(sidx_hbm.at[wid, t], sidx_v)
            pltpu.async_copy(y_hbm.at[sidx_v], rows_v, sem).wait()
            pltpu.sync_copy(didx_hbm.at[wid, t], didx_v)
            pltpu.sync_copy(rows_v, acc.at[didx_v], add=True)
            return carry

        lax.fori_loop(0, NCH, body, 0)
        plsc.subcore_barrier()
        base = c * NPAD + s * RPT
        pltpu.sync_copy(acc.at[pl.ds(s * RPT, RPT)], out_hbm.at[pl.ds(base, RPT)])

    return k


@functools.partial(
    pl.kernel,
    mesh=_mesh,
    out_type=jax.ShapeDtypeStruct((EPAD, NHID2), jnp.float32),
    scratch_types=[
        pltpu.VMEM((CW,), jnp.int32),
        pltpu.VMEM((CW,), jnp.int32),
        pltpu.VMEM((CW, NHID2), jnp.float32),
        pltpu.VMEM((CW, NHID2), jnp.float32),
        pltpu.VMEM((CW, NHID2), jnp.float32),
        pltpu.SemaphoreType.DMA,
        pltpu.SemaphoreType.DMA,
    ],
    compiler_params=_sc_params,
)
def _edge_sum_kernel(a_hbm, b_hbm, uidx_hbm, vidx_hbm, out_hbm,
                     uidx_v, vidx_v, buf_a, buf_b, buf_s, sem_a, sem_b):
    c = lax.axis_index("c")
    s = lax.axis_index("s")
    wid = s * NC + c

    def body(t, carry):
        pltpu.sync_copy(uidx_hbm.at[wid, t], uidx_v)
        pltpu.sync_copy(vidx_hbm.at[wid, t], vidx_v)
        cp_a = pltpu.async_copy(a_hbm.at[uidx_v], buf_a, sem_a)
        cp_b = pltpu.async_copy(b_hbm.at[vidx_v], buf_b, sem_b)
        cp_a.wait()
        cp_b.wait()

        def row(r, cc):
            for j in range(NHID2 // 16):
                sl = pl.ds(j * 16, 16)
                buf_s[r, sl] = buf_a[r, sl] + buf_b[r, sl]
            return cc

        lax.fori_loop(0, CW, row, 0)
        base = (wid * NCH + t) * CW
        pltpu.sync_copy(buf_s, out_hbm.at[pl.ds(base, CW)])
        return carry

    lax.fori_loop(0, NCH, body, 0)


_segsum_deg_128 = _segsum_deg_factory(FDIM)
_segsum_64 = _segsum_factory(NHID2)


def _pad_idx(idx, fill):
    idx = idx.astype(jnp.int32)
    pad = jnp.full((EPAD - E,), fill, jnp.int32)
    return jnp.concatenate([idx, pad]).reshape(NW, NCH, CW)


def _gcn_branch(x, graph, wg1, wg2, z128, z16, z64, ones16):
    src = _pad_idx(graph[0], 0)
    dst = _pad_idx(graph[1], N_NODE)
    y1 = _matmul(x, wg1)
    p, d = _segsum_deg_128(y1, src, dst, z128, z16, ones16)
    y2, dinv = _combine1(p, d, wg2)
    q = _segsum_64(y2, src, dst, z64)
    return q, dinv, src, dst


def kernel(pad_kmers_id_seq, enc_graph, dec_graph, drug_graph, dis_graph,
           drug_sim_feat, disease_sim_feat, Wp, bp, Wd, bd,
           Wg1_drug, Wg2_drug, Wg1_dis, Wg2_dis, Watt, batt, qatt,
           Wdec1, bdec1, Wdec2, bdec2):
    z128 = jnp.zeros((RPT, FDIM), jnp.float32)
    z64 = jnp.zeros((RPT, NHID2), jnp.float32)
    z16 = jnp.zeros((RPT, 16), jnp.float32)
    ones16 = jnp.ones((CW, 16), jnp.float32)

    q_drug, dinv_drug, _, _ = _gcn_branch(
        drug_sim_feat, drug_graph, Wg1_drug, Wg2_drug, z128, z16, z64, ones16)
    q_dis, dinv_dis, _, _ = _gcn_branch(
        disease_sim_feat, dis_graph, Wg1_dis, Wg2_dis, z128, z16, z64, ones16)

    bdec1_2d = bdec1.reshape(1, NHID2)
    zb = jnp.zeros((1, NHID2), jnp.float32)
    sim_drug, a_dec = _combine2(q_drug, dinv_drug, Wdec1[:NHID2], bdec1_2d)
    sim_dis, b_dec = _combine2(q_dis, dinv_dis, Wdec1[NHID2:], zb)

    u = _pad_idx(dec_graph[0], 0)
    v = _pad_idx(dec_graph[1], 0)
    ef = _edge_sum_kernel(a_dec, b_dec, u, v)
    pred = _decoder_mm(ef, Wdec2, bdec2.reshape(1, 2))[:E]

    drug_sim_out = sim_drug[:N_NODE]
    dis_sim_out = sim_dis[:N_NODE]
    return (pred, 0.0, 0.0, drug_sim_out, dis_sim_out)


# speedup vs baseline: 4.0457x; 1.2506x over previous
import functools

import jax
import jax.numpy as jnp
from jax import lax
from jax.experimental import pallas as pl
from jax.experimental.pallas import tpu as pltpu
from jax.experimental.pallas import tpu_sc as plsc

N_NODE = 10000
E = 320000
FDIM = 128
NHID2 = 64

NC = 2
NS = 16
NW = NC * NS
CW = 128
NPAD = 10240
RPT = NPAD // NS

NCHS = 158
ESEG = NS * NCHS * CW
D1 = 80
D2 = 32
NCHD = 80
EDEC = NW * NCHD * CW

_mesh = plsc.VectorSubcoreMesh(core_axis_name="c", subcore_axis_name="s")
_sc_params = pltpu.CompilerParams(use_tc_tiling_on_sc=False)


def _mm_split_body(x_ref, w_ref, o_ref):
    y = jnp.dot(x_ref[...], w_ref[...], preferred_element_type=jnp.float32)
    n = x_ref.shape[0]
    col = lax.broadcasted_iota(jnp.int32, (n, 16), 1)
    deg_block = jnp.where(col == 0, 1.0, 0.0).astype(jnp.float32)
    o_ref[0] = jnp.concatenate([y[:, :NHID2], jnp.zeros((n, 16), jnp.float32)], axis=1)
    o_ref[1] = jnp.concatenate([y[:, NHID2:], deg_block], axis=1)


def _matmul_split(x, w):
    return pl.pallas_call(
        _mm_split_body,
        out_shape=jax.ShapeDtypeStruct((2, NPAD, D1), jnp.float32),
    )(x, w)


def _comb1_body(p_ref, w_ref, y2_ref, dinv_ref):
    o0 = p_ref[:NPAD]
    o1 = p_ref[NPAD:]
    vals = jnp.concatenate([o0[:, :NHID2], o1[:, :NHID2]], axis=1)
    deg = o1[:, NHID2:NHID2 + 1]
    dinv = 1.0 / jnp.maximum(deg, 1.0)
    h = jnp.maximum(vals * dinv, 0.0)
    y2 = jnp.dot(h, w_ref[...], preferred_element_type=jnp.float32)
    y2_ref[0] = y2[:, :D2]
    y2_ref[1] = y2[:, D2:]
    dinv_ref[...] = jnp.broadcast_to(dinv, (NPAD, 16))


def _combine1(p, w):
    return pl.pallas_call(
        _comb1_body,
        out_shape=(
            jax.ShapeDtypeStruct((2, NPAD, D2), jnp.float32),
            jax.ShapeDtypeStruct((NPAD, 16), jnp.float32),
        ),
    )(p, w)


def _comb2_body(q_ref, dinv_ref, w_ref, b_ref, sim_ref, a_ref):
    sim = jnp.concatenate([q_ref[:NPAD], q_ref[NPAD:]], axis=1) * dinv_ref[:, 0:1]
    sim_ref[...] = sim
    a_ref[...] = jnp.dot(sim, w_ref[...], preferred_element_type=jnp.float32) + b_ref[...]


def _combine2(q, dinv, w, b2d):
    return pl.pallas_call(
        _comb2_body,
        out_shape=(
            jax.ShapeDtypeStruct((NPAD, NHID2), jnp.float32),
            jax.ShapeDtypeStruct((NPAD, NHID2), jnp.float32),
        ),
    )(q, dinv, w, b2d)


_DEC_BLK = 4096


def _dec_body(e_ref, w_ref, b_ref, o_ref):
    h = jnp.maximum(e_ref[...], 0.0)
    o_ref[...] = jnp.dot(h, w_ref[...], preferred_element_type=jnp.float32) + b_ref[...]


def _decoder_mm(ef, w, b2d):
    return pl.pallas_call(
        _dec_body,
        grid=(EDEC // _DEC_BLK,),
        in_specs=[
            pl.BlockSpec((_DEC_BLK, NHID2), lambda i: (i, 0)),
            pl.BlockSpec((NHID2, 2), lambda i: (0, 0)),
            pl.BlockSpec((1, 2), lambda i: (0, 0)),
        ],
        out_specs=pl.BlockSpec((_DEC_BLK, 2), lambda i: (i, 0)),
        out_shape=jax.ShapeDtypeStruct((EDEC, 2), jnp.float32),
    )(ef, w, b2d)


def _segsum_factory(D):
    @functools.partial(
        pl.kernel,
        mesh=_mesh,
        out_type=jax.ShapeDtypeStruct((NC * NPAD, D), jnp.float32),
        scratch_types=[
            pltpu.VMEM((NCHS, CW), jnp.int32),
            pltpu.VMEM((NCHS, CW), jnp.int32),
            pltpu.VMEM((CW, D), jnp.float32),
            pltpu.VMEM((CW, D), jnp.float32),
            pltpu.VMEM_SHARED((NPAD, D), jnp.float32),
            pltpu.SemaphoreType.DMA,
            pltpu.SemaphoreType.DMA,
            pltpu.SemaphoreType.DMA,
            pltpu.SemaphoreType.DMA,
        ],
        compiler_params=_sc_params,
    )
    def k(y_hbm, sidx_hbm, didx_hbm, z_hbm, out_hbm,
          sidx, didx, rows_a, rows_b, acc, ga, gb, sa, sb):
        c = lax.axis_index("c")
        s = lax.axis_index("s")
        pltpu.sync_copy(sidx_hbm.at[c, s], sidx)
        pltpu.sync_copy(didx_hbm.at[s], didx)
        pltpu.async_copy(y_hbm.at[sidx.at[0]], rows_a, ga)
        pltpu.async_copy(y_hbm.at[sidx.at[1]], rows_b, gb)
        pltpu.sync_copy(z_hbm, acc.at[pl.ds(s * RPT, RPT)])
        plsc.subcore_barrier()

        def body(i, carry):
            t0 = 2 * i
            t1 = t0 + 1
            pltpu.make_async_copy(y_hbm.at[sidx.at[t0]], rows_a, ga).wait()
            pltpu.async_copy(rows_a, acc.at[didx.at[t0]], sa, add=True)
            pltpu.make_async_copy(y_hbm.at[sidx.at[t1]], rows_b, gb).wait()
            pltpu.async_copy(rows_b, acc.at[didx.at[t1]], sb, add=True)
            pltpu.make_async_copy(rows_a, acc.at[didx.at[t0]], sa).wait()

            @pl.when(t0 + 2 < NCHS)
            def _():
                pltpu.async_copy(y_hbm.at[sidx.at[t0 + 2]], rows_a, ga)

            pltpu.make_async_copy(rows_b, acc.at[didx.at[t1]], sb).wait()

            @pl.when(t1 + 2 < NCHS)
            def _():
                pltpu.async_copy(y_hbm.at[sidx.at[t1 + 2]], rows_b, gb)

            return carry

        lax.fori_loop(0, NCHS // 2, body, 0)
        plsc.subcore_barrier()
        base = c * NPAD + s * RPT
        pltpu.sync_copy(acc.at[pl.ds(s * RPT, RPT)], out_hbm.at[pl.ds(base, RPT)])

    return k


@functools.partial(
    pl.kernel,
    mesh=_mesh,
    out_type=jax.ShapeDtypeStruct((EDEC, NHID2), jnp.float32),
    scratch_types=[
        pltpu.VMEM((NCHD, CW), jnp.int32),
        pltpu.VMEM((NCHD, CW), jnp.int32),
        pltpu.VMEM((CW, NHID2), jnp.float32),
        pltpu.VMEM((CW, NHID2), jnp.float32),
        pltpu.VMEM((CW, NHID2), jnp.float32),
        pltpu.VMEM((CW, NHID2), jnp.float32),
        pltpu.VMEM((CW, NHID2), jnp.float32),
        pltpu.VMEM((CW, NHID2), jnp.float32),
        pltpu.SemaphoreType.DMA,
        pltpu.SemaphoreType.DMA,
        pltpu.SemaphoreType.DMA,
        pltpu.SemaphoreType.DMA,
        pltpu.SemaphoreType.DMA,
        pltpu.SemaphoreType.DMA,
    ],
    compiler_params=_sc_params,
)
def _edge_sum_kernel(a_hbm, b_hbm, uidx_hbm, vidx_hbm, out_hbm,
                     uidx, vidx, a0, b0, a1, b1, s0, s1,
                     sa0, sb0, sa1, sb1, ss0, ss1):
    c = lax.axis_index("c")
    s = lax.axis_index("s")
    wid = s * NC + c
    pltpu.sync_copy(uidx_hbm.at[wid], uidx)
    pltpu.sync_copy(vidx_hbm.at[wid], vidx)

    def _add(src_a, src_b, dst):
        def row(r, cc):
            for rr in range(4):
                for j in range(NHID2 // 16):
                    sl = pl.ds(j * 16, 16)
                    dst[r * 4 + rr, sl] = src_a[r * 4 + rr, sl] + src_b[r * 4 + rr, sl]
            return cc

        lax.fori_loop(0, CW // 4, row, 0)

    def _out_slice(t):
        return out_hbm.at[pl.ds((wid * NCHD + t) * CW, CW)]

    pltpu.async_copy(a_hbm.at[uidx.at[0]], a0, sa0)
    pltpu.async_copy(b_hbm.at[vidx.at[0]], b0, sb0)
    pltpu.async_copy(a_hbm.at[uidx.at[1]], a1, sa1)
    pltpu.async_copy(b_hbm.at[vidx.at[1]], b1, sb1)

    def body(i, carry):
        t0 = 2 * i
        t1 = t0 + 1
        pltpu.make_async_copy(a_hbm.at[uidx.at[t0]], a0, sa0).wait()
        pltpu.make_async_copy(b_hbm.at[vidx.at[t0]], b0, sb0).wait()

        @pl.when(i > 0)
        def _():
            pltpu.make_async_copy(s0, _out_slice(t0 - 2), ss0).wait()

        _add(a0, b0, s0)
        pltpu.async_copy(s0, _out_slice(t0), ss0)

        @pl.when(t0 + 2 < NCHD)
        def _():
            pltpu.async_copy(a_hbm.at[uidx.at[t0 + 2]], a0, sa0)
            pltpu.async_copy(b_hbm.at[vidx.at[t0 + 2]], b0, sb0)

        pltpu.make_async_copy(a_hbm.at[uidx.at[t1]], a1, sa1).wait()
        pltpu.make_async_copy(b_hbm.at[vidx.at[t1]], b1, sb1).wait()

        @pl.when(i > 0)
        def _():
            pltpu.make_async_copy(s1, _out_slice(t1 - 2), ss1).wait()

        _add(a1, b1, s1)
        pltpu.async_copy(s1, _out_slice(t1), ss1)

        @pl.when(t1 + 2 < NCHD)
        def _():
            pltpu.async_copy(a_hbm.at[uidx.at[t1 + 2]], a1, sa1)
            pltpu.async_copy(b_hbm.at[vidx.at[t1 + 2]], b1, sb1)

        return carry

    lax.fori_loop(0, NCHD // 2, body, 0)
    pltpu.make_async_copy(s0, _out_slice(NCHD - 2), ss0).wait()
    pltpu.make_async_copy(s1, _out_slice(NCHD - 1), ss1).wait()


_segsum_d1 = _segsum_factory(D1)
_segsum_d2 = _segsum_factory(D2)


def _pad_to(idx, n, fill):
    idx = idx.astype(jnp.int32)
    pad = jnp.full((n - E,), fill, jnp.int32)
    return jnp.concatenate([idx, pad])


def _seg_idx(graph):
    src = _pad_to(graph[0], ESEG, 0).reshape(NS, NCHS, CW)
    src_st = jnp.stack([src, src + NPAD])
    dst = _pad_to(graph[1], ESEG, N_NODE).reshape(NS, NCHS, CW)
    return src_st, dst


def _gcn_branch(x_pad, src_st, dst, wg1, wg2, z80, z32):
    y1 = _matmul_split(x_pad, wg1)
    p = _segsum_d1(y1.reshape(2 * NPAD, D1), src_st, dst, z80)
    y2, dinv = _combine1(p, wg2)
    q = _segsum_d2(y2.reshape(2 * NPAD, D2), src_st, dst, z32)
    return q, dinv


def kernel(pad_kmers_id_seq, enc_graph, dec_graph, drug_graph, dis_graph,
           drug_sim_feat, disease_sim_feat, Wp, bp, Wd, bd,
           Wg1_drug, Wg2_drug, Wg1_dis, Wg2_dis, Watt, batt, qatt,
           Wdec1, bdec1, Wdec2, bdec2):
    z80 = jnp.zeros((RPT, D1), jnp.float32)
    z32 = jnp.zeros((RPT, D2), jnp.float32)
    xpad_drug = jnp.pad(drug_sim_feat, ((0, NPAD - N_NODE), (0, 0)))
    xpad_dis = jnp.pad(disease_sim_feat, ((0, NPAD - N_NODE), (0, 0)))

    src_drug, dst_drug = _seg_idx(drug_graph)
    src_dis, dst_dis = _seg_idx(dis_graph)
    q_drug, dinv_drug = _gcn_branch(xpad_drug, src_drug, dst_drug,
                                    Wg1_drug, Wg2_drug, z80, z32)
    q_dis, dinv_dis = _gcn_branch(xpad_dis, src_dis, dst_dis,
                                  Wg1_dis, Wg2_dis, z80, z32)

    bdec1_2d = bdec1.reshape(1, NHID2)
    zb = jnp.zeros((1, NHID2), jnp.float32)
    sim_drug, a_dec = _combine2(q_drug.reshape(2 * NPAD, D2), dinv_drug,
                                Wdec1[:NHID2], bdec1_2d)
    sim_dis, b_dec = _combine2(q_dis.reshape(2 * NPAD, D2), dinv_dis,
                               Wdec1[NHID2:], zb)

    u = _pad_to(dec_graph[0], EDEC, 0).reshape(NW, NCHD, CW)
    v = _pad_to(dec_graph[1], EDEC, 0).reshape(NW, NCHD, CW)
    ef = _edge_sum_kernel(a_dec, b_dec, u, v)
    pred = _decoder_mm(ef, Wdec2, bdec2.reshape(1, 2))[:E]

    drug_sim_out = sim_drug[:N_NODE]
    dis_sim_out = sim_dis[:N_NODE]
    return (pred, 0.0, 0.0, drug_sim_out, dis_sim_out)


# speedup vs baseline: 4.2604x; 1.0531x over previous
import functools

import jax
import jax.numpy as jnp
from jax import lax
from jax.experimental import pallas as pl
from jax.experimental.pallas import tpu as pltpu
from jax.experimental.pallas import tpu_sc as plsc

N_NODE = 10000
E = 320000
FDIM = 128
NHID2 = 64

NC = 2
NS = 16
NW = NC * NS
CW = 128
NPAD = 10240
RPT = NPAD // NS

NCHS = 158
ESEG = NS * NCHS * CW
D1 = 64
D2 = 32
DDEG = 16
NCHD = 80
EDEC = NW * NCHD * CW

_mesh = plsc.VectorSubcoreMesh(core_axis_name="c", subcore_axis_name="s")
_sc_params = pltpu.CompilerParams(use_tc_tiling_on_sc=False)


def _mm_split_body(x_ref, w_ref, o_ref):
    y = jnp.dot(x_ref[...], w_ref[...], preferred_element_type=jnp.float32)
    o_ref[0] = y[:, :NHID2]
    o_ref[1] = y[:, NHID2:]


def _matmul_split(x, w):
    return pl.pallas_call(
        _mm_split_body,
        out_shape=jax.ShapeDtypeStruct((2, NPAD, D1), jnp.float32),
    )(x, w)


def _comb1_body(p_ref, deg_ref, w_ref, y2_ref, dinv_ref):
    vals = jnp.concatenate([p_ref[:NPAD], p_ref[NPAD:]], axis=1)
    deg = deg_ref[:, 0:1]
    dinv = 1.0 / jnp.maximum(deg, 1.0)
    h = jnp.maximum(vals * dinv, 0.0)
    y2 = jnp.dot(h, w_ref[...], preferred_element_type=jnp.float32)
    y2_ref[0] = y2[:, :D2]
    y2_ref[1] = y2[:, D2:]
    dinv_ref[...] = jnp.broadcast_to(dinv, (NPAD, 16))


def _combine1(p, deg, w):
    return pl.pallas_call(
        _comb1_body,
        out_shape=(
            jax.ShapeDtypeStruct((2, NPAD, D2), jnp.float32),
            jax.ShapeDtypeStruct((NPAD, 16), jnp.float32),
        ),
    )(p, deg, w)


def _comb2_body(q_ref, dinv_ref, w_ref, b_ref, sim_ref, a_ref):
    sim = jnp.concatenate([q_ref[:NPAD], q_ref[NPAD:]], axis=1) * dinv_ref[:, 0:1]
    sim_ref[...] = sim
    a_ref[...] = jnp.dot(sim, w_ref[...], preferred_element_type=jnp.float32) + b_ref[...]


def _combine2(q, dinv, w, b2d):
    return pl.pallas_call(
        _comb2_body,
        out_shape=(
            jax.ShapeDtypeStruct((NPAD, NHID2), jnp.float32),
            jax.ShapeDtypeStruct((NPAD, NHID2), jnp.float32),
        ),
    )(q, dinv, w, b2d)


_DEC_BLK = 4096


def _dec_body(e_ref, w_ref, b_ref, o_ref):
    h = jnp.maximum(e_ref[...], 0.0)
    o_ref[...] = jnp.dot(h, w_ref[...], preferred_element_type=jnp.float32) + b_ref[...]


def _decoder_mm(ef, w, b2d):
    return pl.pallas_call(
        _dec_body,
        grid=(EDEC // _DEC_BLK,),
        in_specs=[
            pl.BlockSpec((_DEC_BLK, NHID2), lambda i: (i, 0)),
            pl.BlockSpec((NHID2, 2), lambda i: (0, 0)),
            pl.BlockSpec((1, 2), lambda i: (0, 0)),
        ],
        out_specs=pl.BlockSpec((_DEC_BLK, 2), lambda i: (i, 0)),
        out_shape=jax.ShapeDtypeStruct((EDEC, 2), jnp.float32),
    )(ef, w, b2d)


def _segsum_factory(D, with_deg):
    if with_deg:
        out_type = (
            jax.ShapeDtypeStruct((NC * NPAD, D), jnp.float32),
            jax.ShapeDtypeStruct((NPAD, DDEG), jnp.float32),
        )
    else:
        out_type = jax.ShapeDtypeStruct((NC * NPAD, D), jnp.float32)

    scratch = [
        pltpu.VMEM((NCHS, CW), jnp.int32),
        pltpu.VMEM((NCHS, CW), jnp.int32),
        pltpu.VMEM((CW, D), jnp.float32),
        pltpu.VMEM((CW, D), jnp.float32),
        pltpu.VMEM_SHARED((NPAD, D), jnp.float32),
        pltpu.SemaphoreType.DMA,
        pltpu.SemaphoreType.DMA,
        pltpu.SemaphoreType.DMA,
        pltpu.SemaphoreType.DMA,
    ]
    if with_deg:
        scratch += [
            pltpu.VMEM((CW, DDEG), jnp.float32),
            pltpu.VMEM_SHARED((NPAD, DDEG), jnp.float32),
            pltpu.SemaphoreType.DMA,
            pltpu.SemaphoreType.DMA,
        ]

    @functools.partial(
        pl.kernel,
        mesh=_mesh,
        out_type=out_type,
        scratch_types=scratch,
        compiler_params=_sc_params,
    )
    def k(*refs):
        if with_deg:
            (y_hbm, sidx_hbm, didx_hbm, z_hbm, zdeg_hbm, ones_hbm,
             out_hbm, degout_hbm,
             sidx, didx, rows_a, rows_b, acc, ga, gb, sa, sb,
             ones, dacc, da, db) = refs
        else:
            (y_hbm, sidx_hbm, didx_hbm, z_hbm,
             out_hbm,
             sidx, didx, rows_a, rows_b, acc, ga, gb, sa, sb) = refs
        c = lax.axis_index("c")
        s = lax.axis_index("s")
        pltpu.sync_copy(sidx_hbm.at[c, s], sidx)
        pltpu.sync_copy(didx_hbm.at[s], didx)
        pltpu.async_copy(y_hbm.at[sidx.at[0]], rows_a, ga)
        pltpu.async_copy(y_hbm.at[sidx.at[1]], rows_b, gb)
        pltpu.sync_copy(z_hbm, acc.at[pl.ds(s * RPT, RPT)])
        if with_deg:
            pltpu.sync_copy(ones_hbm, ones)
            pltpu.sync_copy(zdeg_hbm, dacc.at[pl.ds(s * RPT, RPT)])
        plsc.subcore_barrier()

        def body(i, carry):
            t0 = 2 * i
            t1 = t0 + 1
            pltpu.make_async_copy(y_hbm.at[sidx.at[t0]], rows_a, ga).wait()
            pltpu.async_copy(rows_a, acc.at[didx.at[t0]], sa, add=True)
            if with_deg:
                @pl.when(c == 1)
                def _():
                    pltpu.async_copy(ones, dacc.at[didx.at[t0]], da, add=True)
            pltpu.make_async_copy(y_hbm.at[sidx.at[t1]], rows_b, gb).wait()
            pltpu.async_copy(rows_b, acc.at[didx.at[t1]], sb, add=True)
            if with_deg:
                @pl.when(c == 1)
                def _():
                    pltpu.async_copy(ones, dacc.at[didx.at[t1]], db, add=True)
            pltpu.make_async_copy(rows_a, acc.at[didx.at[t0]], sa).wait()

            @pl.when(t0 + 2 < NCHS)
            def _():
                pltpu.async_copy(y_hbm.at[sidx.at[t0 + 2]], rows_a, ga)

            pltpu.make_async_copy(rows_b, acc.at[didx.at[t1]], sb).wait()

            @pl.when(t1 + 2 < NCHS)
            def _():
                pltpu.async_copy(y_hbm.at[sidx.at[t1 + 2]], rows_b, gb)

            if with_deg:
                @pl.when(c == 1)
                def _():
                    pltpu.make_async_copy(ones, dacc.at[didx.at[t0]], da).wait()
                    pltpu.make_async_copy(ones, dacc.at[didx.at[t1]], db).wait()

            return carry

        lax.fori_loop(0, NCHS // 2, body, 0)
        plsc.subcore_barrier()
        base = c * NPAD + s * RPT
        pltpu.sync_copy(acc.at[pl.ds(s * RPT, RPT)], out_hbm.at[pl.ds(base, RPT)])
        if with_deg:
            @pl.when(c == 1)
            def _():
                pltpu.sync_copy(dacc.at[pl.ds(s * RPT, RPT)],
                                degout_hbm.at[pl.ds(s * RPT, RPT)])

    return k


@functools.partial(
    pl.kernel,
    mesh=_mesh,
    out_type=jax.ShapeDtypeStruct((EDEC, NHID2), jnp.float32),
    scratch_types=[
        pltpu.VMEM((NCHD, CW), jnp.int32),
        pltpu.VMEM((NCHD, CW), jnp.int32),
        pltpu.VMEM((CW, NHID2), jnp.float32),
        pltpu.VMEM((CW, NHID2), jnp.float32),
        pltpu.VMEM((CW, NHID2), jnp.float32),
        pltpu.VMEM((CW, NHID2), jnp.float32),
        pltpu.VMEM((CW, NHID2), jnp.float32),
        pltpu.VMEM((CW, NHID2), jnp.float32),
        pltpu.SemaphoreType.DMA,
        pltpu.SemaphoreType.DMA,
        pltpu.SemaphoreType.DMA,
        pltpu.SemaphoreType.DMA,
        pltpu.SemaphoreType.DMA,
        pltpu.SemaphoreType.DMA,
    ],
    compiler_params=_sc_params,
)
def _edge_sum_kernel(a_hbm, b_hbm, uidx_hbm, vidx_hbm, out_hbm,
                     uidx, vidx, a0, b0, a1, b1, s0, s1,
                     sa0, sb0, sa1, sb1, ss0, ss1):
    c = lax.axis_index("c")
    s = lax.axis_index("s")
    wid = s * NC + c
    pltpu.sync_copy(uidx_hbm.at[wid], uidx)
    pltpu.sync_copy(vidx_hbm.at[wid], vidx)

    def _add(src_a, src_b, dst):
        def row(r, cc):
            for rr in range(4):
                for j in range(NHID2 // 16):
                    sl = pl.ds(j * 16, 16)
                    dst[r * 4 + rr, sl] = src_a[r * 4 + rr, sl] + src_b[r * 4 + rr, sl]
            return cc

        lax.fori_loop(0, CW // 4, row, 0)

    def _out_slice(t):
        return out_hbm.at[pl.ds((wid * NCHD + t) * CW, CW)]

    pltpu.async_copy(a_hbm.at[uidx.at[0]], a0, sa0)
    pltpu.async_copy(b_hbm.at[vidx.at[0]], b0, sb0)
    pltpu.async_copy(a_hbm.at[uidx.at[1]], a1, sa1)
    pltpu.async_copy(b_hbm.at[vidx.at[1]], b1, sb1)

    def body(i, carry):
        t0 = 2 * i
        t1 = t0 + 1
        pltpu.make_async_copy(a_hbm.at[uidx.at[t0]], a0, sa0).wait()
        pltpu.make_async_copy(b_hbm.at[vidx.at[t0]], b0, sb0).wait()

        @pl.when(i > 0)
        def _():
            pltpu.make_async_copy(s0, _out_slice(t0 - 2), ss0).wait()

        _add(a0, b0, s0)
        pltpu.async_copy(s0, _out_slice(t0), ss0)

        @pl.when(t0 + 2 < NCHD)
        def _():
            pltpu.async_copy(a_hbm.at[uidx.at[t0 + 2]], a0, sa0)
            pltpu.async_copy(b_hbm.at[vidx.at[t0 + 2]], b0, sb0)

        pltpu.make_async_copy(a_hbm.at[uidx.at[t1]], a1, sa1).wait()
        pltpu.make_async_copy(b_hbm.at[vidx.at[t1]], b1, sb1).wait()

        @pl.when(i > 0)
        def _():
            pltpu.make_async_copy(s1, _out_slice(t1 - 2), ss1).wait()

        _add(a1, b1, s1)
        pltpu.async_copy(s1, _out_slice(t1), ss1)

        @pl.when(t1 + 2 < NCHD)
        def _():
            pltpu.async_copy(a_hbm.at[uidx.at[t1 + 2]], a1, sa1)
            pltpu.async_copy(b_hbm.at[vidx.at[t1 + 2]], b1, sb1)

        return carry

    lax.fori_loop(0, NCHD // 2, body, 0)
    pltpu.make_async_copy(s0, _out_slice(NCHD - 2), ss0).wait()
    pltpu.make_async_copy(s1, _out_slice(NCHD - 1), ss1).wait()


_segsum_d1 = _segsum_factory(D1, with_deg=True)
_segsum_d2 = _segsum_factory(D2, with_deg=False)


def _pad_to(idx, n, fill):
    idx = idx.astype(jnp.int32)
    pad = jnp.full((n - E,), fill, jnp.int32)
    return jnp.concatenate([idx, pad])


def _seg_idx(graph):
    src = _pad_to(graph[0], ESEG, 0).reshape(NS, NCHS, CW)
    src_st = jnp.stack([src, src + NPAD])
    dst = _pad_to(graph[1], ESEG, N_NODE).reshape(NS, NCHS, CW)
    return src_st, dst


def _gcn_branch(x_pad, src_st, dst, wg1, wg2, z64, z32, z16, ones16):
    y1 = _matmul_split(x_pad, wg1)
    p, deg = _segsum_d1(y1.reshape(2 * NPAD, D1), src_st, dst, z64, z16, ones16)
    y2, dinv = _combine1(p, deg, wg2)
    q = _segsum_d2(y2.reshape(2 * NPAD, D2), src_st, dst, z32)
    return q, dinv


def kernel(pad_kmers_id_seq, enc_graph, dec_graph, drug_graph, dis_graph,
           drug_sim_feat, disease_sim_feat, Wp, bp, Wd, bd,
           Wg1_drug, Wg2_drug, Wg1_dis, Wg2_dis, Watt, batt, qatt,
           Wdec1, bdec1, Wdec2, bdec2):
    z64 = jnp.zeros((RPT, D1), jnp.float32)
    z32 = jnp.zeros((RPT, D2), jnp.float32)
    z16 = jnp.zeros((RPT, DDEG), jnp.float32)
    ones16 = jnp.ones((CW, DDEG), jnp.float32)
    xpad_drug = jnp.pad(drug_sim_feat, ((0, NPAD - N_NODE), (0, 0)))
    xpad_dis = jnp.pad(disease_sim_feat, ((0, NPAD - N_NODE), (0, 0)))

    src_drug, dst_drug = _seg_idx(drug_graph)
    src_dis, dst_dis = _seg_idx(dis_graph)
    q_drug, dinv_drug = _gcn_branch(xpad_drug, src_drug, dst_drug,
                                    Wg1_drug, Wg2_drug, z64, z32, z16, ones16)
    q_dis, dinv_dis = _gcn_branch(xpad_dis, src_dis, dst_dis,
                                  Wg1_dis, Wg2_dis, z64, z32, z16, ones16)

    bdec1_2d = bdec1.reshape(1, NHID2)
    zb = jnp.zeros((1, NHID2), jnp.float32)
    sim_drug, a_dec = _combine2(q_drug.reshape(2 * NPAD, D2), dinv_drug,
                                Wdec1[:NHID2], bdec1_2d)
    sim_dis, b_dec = _combine2(q_dis.reshape(2 * NPAD, D2), dinv_dis,
                               Wdec1[NHID2:], zb)

    u = _pad_to(dec_graph[0], EDEC, 0).reshape(NW, NCHD, CW)
    v = _pad_to(dec_graph[1], EDEC, 0).reshape(NW, NCHD, CW)
    ef = _edge_sum_kernel(a_dec, b_dec, u, v)
    pred = _decoder_mm(ef, Wdec2, bdec2.reshape(1, 2))[:E]

    drug_sim_out = sim_drug[:N_NODE]
    dis_sim_out = sim_dis[:N_NODE]
    return (pred, 0.0, 0.0, drug_sim_out, dis_sim_out)


# speedup vs baseline: 5.2214x; 1.2256x over previous
import functools

import jax
import jax.numpy as jnp
from jax import lax
from jax.experimental import pallas as pl
from jax.experimental.pallas import tpu as pltpu
from jax.experimental.pallas import tpu_sc as plsc

N_NODE = 10000
E = 320000
FDIM = 128
NHID2 = 64

NC = 2
NS = 16
NW = NC * NS
CW = 128
NPAD = 10240
RPT = NPAD // NS

NCHS = 158
ESEG = NS * NCHS * CW
D1 = 64
D2 = 32
DDEG = 16
NCHD = 80
EDEC = NW * NCHD * CW

_mesh = plsc.VectorSubcoreMesh(core_axis_name="c", subcore_axis_name="s")
_sc_params = pltpu.CompilerParams(use_tc_tiling_on_sc=False)


def _mm_split_body(x_ref, w_ref, o_ref):
    y = jnp.dot(x_ref[...], w_ref[...], preferred_element_type=jnp.float32)
    o_ref[0] = y[:, :NHID2]
    o_ref[1] = y[:, NHID2:]


def _matmul_split(x, w):
    return pl.pallas_call(
        _mm_split_body,
        out_shape=jax.ShapeDtypeStruct((2, NPAD, D1), jnp.float32),
    )(x, w)


def _comb1_body(p_ref, deg_ref, w_ref, y2_ref, dinv_ref):
    vals = jnp.concatenate([p_ref[:NPAD], p_ref[NPAD:]], axis=1)
    deg = deg_ref[:, 0:1]
    dinv = 1.0 / jnp.maximum(deg, 1.0)
    h = jnp.maximum(vals * dinv, 0.0)
    y2 = jnp.dot(h, w_ref[...], preferred_element_type=jnp.float32)
    y2_ref[0] = y2[:, :D2]
    y2_ref[1] = y2[:, D2:]
    dinv_ref[...] = jnp.broadcast_to(dinv, (NPAD, 16))


def _combine1(p, deg, w):
    return pl.pallas_call(
        _comb1_body,
        out_shape=(
            jax.ShapeDtypeStruct((2, NPAD, D2), jnp.float32),
            jax.ShapeDtypeStruct((NPAD, 16), jnp.float32),
        ),
    )(p, deg, w)


def _comb2_body(q_ref, dinv_ref, w_ref, b_ref, sim_ref, a_ref):
    sim = jnp.concatenate([q_ref[:NPAD], q_ref[NPAD:]], axis=1) * dinv_ref[:, 0:1]
    sim_ref[...] = sim
    a_ref[...] = jnp.dot(sim, w_ref[...], preferred_element_type=jnp.float32) + b_ref[...]


def _combine2(q, dinv, w, b2d):
    return pl.pallas_call(
        _comb2_body,
        out_shape=(
            jax.ShapeDtypeStruct((NPAD, NHID2), jnp.float32),
            jax.ShapeDtypeStruct((NPAD, NHID2), jnp.float32),
        ),
    )(q, dinv, w, b2d)


_DEC_BLK = 4096


def _dec_body(e_ref, w_ref, b_ref, o_ref):
    h = jnp.maximum(e_ref[...], 0.0)
    o_ref[...] = jnp.dot(h, w_ref[...], preferred_element_type=jnp.float32) + b_ref[...]


def _decoder_mm(ef, w, b2d):
    return pl.pallas_call(
        _dec_body,
        grid=(EDEC // _DEC_BLK,),
        in_specs=[
            pl.BlockSpec((_DEC_BLK, NHID2), lambda i: (i, 0)),
            pl.BlockSpec((NHID2, 2), lambda i: (0, 0)),
            pl.BlockSpec((1, 2), lambda i: (0, 0)),
        ],
        out_specs=pl.BlockSpec((_DEC_BLK, 2), lambda i: (i, 0)),
        out_shape=jax.ShapeDtypeStruct((EDEC, 2), jnp.float32),
    )(ef, w, b2d)


def _segsum_factory(D):
    @functools.partial(
        pl.kernel,
        mesh=_mesh,
        out_type=jax.ShapeDtypeStruct((NC * NPAD, D), jnp.float32),
        scratch_types=[
            pltpu.VMEM((NCHS, CW), jnp.int32),
            pltpu.VMEM((2, CW), jnp.int32),
            pltpu.VMEM((CW, D), jnp.float32),
            pltpu.VMEM((CW, D), jnp.float32),
            pltpu.VMEM_SHARED((NPAD, D), jnp.float32),
            pltpu.VMEM_SHARED((NPAD, D), jnp.float32),
            pltpu.SemaphoreType.DMA,
            pltpu.SemaphoreType.DMA,
            pltpu.SemaphoreType.DMA,
            pltpu.SemaphoreType.DMA,
            pltpu.SemaphoreType.DMA,
            pltpu.SemaphoreType.DMA,
        ],
        compiler_params=_sc_params,
    )
    def k(y_hbm, sidx_hbm, didx_hbm, z_hbm, out_hbm,
          sidx, dbuf, rows_a, rows_b, acc, ytab, ga, gb, sa, sb, d0, d1):
        c = lax.axis_index("c")
        s = lax.axis_index("s")
        pltpu.sync_copy(sidx_hbm.at[s], sidx)
        pltpu.sync_copy(y_hbm.at[pl.ds(c * NPAD + s * RPT, RPT)],
                        ytab.at[pl.ds(s * RPT, RPT)])
        pltpu.sync_copy(z_hbm, acc.at[pl.ds(s * RPT, RPT)])
        plsc.subcore_barrier()
        pltpu.async_copy(ytab.at[sidx.at[0]], rows_a, ga)
        pltpu.async_copy(ytab.at[sidx.at[1]], rows_b, gb)
        pltpu.async_copy(didx_hbm.at[s, 0], dbuf.at[0], d0)
        pltpu.async_copy(didx_hbm.at[s, 1], dbuf.at[1], d1)

        def body(i, carry):
            t0 = 2 * i
            t1 = t0 + 1
            pltpu.make_async_copy(ytab.at[sidx.at[t0]], rows_a, ga).wait()
            pltpu.make_async_copy(didx_hbm.at[s, t0], dbuf.at[0], d0).wait()
            pltpu.async_copy(rows_a, acc.at[dbuf.at[0]], sa, add=True)
            pltpu.make_async_copy(ytab.at[sidx.at[t1]], rows_b, gb).wait()
            pltpu.make_async_copy(didx_hbm.at[s, t1], dbuf.at[1], d1).wait()
            pltpu.async_copy(rows_b, acc.at[dbuf.at[1]], sb, add=True)
            pltpu.make_async_copy(rows_a, acc.at[dbuf.at[0]], sa).wait()

            @pl.when(t0 + 2 < NCHS)
            def _():
                pltpu.async_copy(ytab.at[sidx.at[t0 + 2]], rows_a, ga)
                pltpu.async_copy(didx_hbm.at[s, t0 + 2], dbuf.at[0], d0)

            pltpu.make_async_copy(rows_b, acc.at[dbuf.at[1]], sb).wait()

            @pl.when(t1 + 2 < NCHS)
            def _():
                pltpu.async_copy(ytab.at[sidx.at[t1 + 2]], rows_b, gb)
                pltpu.async_copy(didx_hbm.at[s, t1 + 2], dbuf.at[1], d1)

            return carry

        lax.fori_loop(0, NCHS // 2, body, 0)
        plsc.subcore_barrier()
        base = c * NPAD + s * RPT
        pltpu.sync_copy(acc.at[pl.ds(s * RPT, RPT)], out_hbm.at[pl.ds(base, RPT)])

    return k


@functools.partial(
    pl.kernel,
    mesh=_mesh,
    out_type=jax.ShapeDtypeStruct((NC, NPAD, DDEG), jnp.float32),
    scratch_types=[
        pltpu.VMEM((NCHS, CW), jnp.int32),
        pltpu.VMEM((CW, DDEG), jnp.float32),
        pltpu.VMEM_SHARED((NPAD, DDEG), jnp.float32),
        pltpu.SemaphoreType.DMA,
        pltpu.SemaphoreType.DMA,
    ],
    compiler_params=_sc_params,
)
def _deg_kernel(didx_hbm, ones_hbm, zdeg_hbm, out_hbm,
                didx, ones, dacc, da, db):
    c = lax.axis_index("c")
    s = lax.axis_index("s")
    pltpu.sync_copy(didx_hbm.at[c, s], didx)
    pltpu.sync_copy(ones_hbm, ones)
    pltpu.sync_copy(zdeg_hbm, dacc.at[pl.ds(s * RPT, RPT)])
    plsc.subcore_barrier()

    def body(i, carry):
        t0 = 2 * i
        t1 = t0 + 1
        pltpu.async_copy(ones, dacc.at[didx.at[t0]], da, add=True)
        pltpu.async_copy(ones, dacc.at[didx.at[t1]], db, add=True)
        pltpu.make_async_copy(ones, dacc.at[didx.at[t0]], da).wait()
        pltpu.make_async_copy(ones, dacc.at[didx.at[t1]], db).wait()
        return carry

    lax.fori_loop(0, NCHS // 2, body, 0)
    plsc.subcore_barrier()
    pltpu.sync_copy(dacc.at[pl.ds(s * RPT, RPT)],
                    out_hbm.at[c, pl.ds(s * RPT, RPT)])


@functools.partial(
    pl.kernel,
    mesh=_mesh,
    out_type=jax.ShapeDtypeStruct((EDEC, NHID2), jnp.float32),
    scratch_types=[
        pltpu.VMEM((NCHD, CW), jnp.int32),
        pltpu.VMEM((NCHD, CW), jnp.int32),
        pltpu.VMEM((CW, NHID2), jnp.float32),
        pltpu.VMEM((CW, NHID2), jnp.float32),
        pltpu.VMEM((CW, NHID2), jnp.float32),
        pltpu.VMEM((CW, NHID2), jnp.float32),
        pltpu.VMEM((CW, NHID2), jnp.float32),
        pltpu.VMEM((CW, NHID2), jnp.float32),
        pltpu.SemaphoreType.DMA,
        pltpu.SemaphoreType.DMA,
        pltpu.SemaphoreType.DMA,
        pltpu.SemaphoreType.DMA,
        pltpu.SemaphoreType.DMA,
        pltpu.SemaphoreType.DMA,
    ],
    compiler_params=_sc_params,
)
def _edge_sum_kernel(a_hbm, b_hbm, uidx_hbm, vidx_hbm, out_hbm,
                     uidx, vidx, a0, b0, a1, b1, s0, s1,
                     sa0, sb0, sa1, sb1, ss0, ss1):
    c = lax.axis_index("c")
    s = lax.axis_index("s")
    wid = s * NC + c
    pltpu.sync_copy(uidx_hbm.at[wid], uidx)
    pltpu.sync_copy(vidx_hbm.at[wid], vidx)

    def _add(src_a, src_b, dst):
        def row(r, cc):
            for rr in range(4):
                for j in range(NHID2 // 16):
                    sl = pl.ds(j * 16, 16)
                    dst[r * 4 + rr, sl] = src_a[r * 4 + rr, sl] + src_b[r * 4 + rr, sl]
            return cc

        lax.fori_loop(0, CW // 4, row, 0)

    def _out_slice(t):
        return out_hbm.at[pl.ds((wid * NCHD + t) * CW, CW)]

    pltpu.async_copy(a_hbm.at[uidx.at[0]], a0, sa0)
    pltpu.async_copy(b_hbm.at[vidx.at[0]], b0, sb0)
    pltpu.async_copy(a_hbm.at[uidx.at[1]], a1, sa1)
    pltpu.async_copy(b_hbm.at[vidx.at[1]], b1, sb1)

    def body(i, carry):
        t0 = 2 * i
        t1 = t0 + 1
        pltpu.make_async_copy(a_hbm.at[uidx.at[t0]], a0, sa0).wait()
        pltpu.make_async_copy(b_hbm.at[vidx.at[t0]], b0, sb0).wait()

        @pl.when(i > 0)
        def _():
            pltpu.make_async_copy(s0, _out_slice(t0 - 2), ss0).wait()

        _add(a0, b0, s0)
        pltpu.async_copy(s0, _out_slice(t0), ss0)

        @pl.when(t0 + 2 < NCHD)
        def _():
            pltpu.async_copy(a_hbm.at[uidx.at[t0 + 2]], a0, sa0)
            pltpu.async_copy(b_hbm.at[vidx.at[t0 + 2]], b0, sb0)

        pltpu.make_async_copy(a_hbm.at[uidx.at[t1]], a1, sa1).wait()
        pltpu.make_async_copy(b_hbm.at[vidx.at[t1]], b1, sb1).wait()

        @pl.when(i > 0)
        def _():
            pltpu.make_async_copy(s1, _out_slice(t1 - 2), ss1).wait()

        _add(a1, b1, s1)
        pltpu.async_copy(s1, _out_slice(t1), ss1)

        @pl.when(t1 + 2 < NCHD)
        def _():
            pltpu.async_copy(a_hbm.at[uidx.at[t1 + 2]], a1, sa1)
            pltpu.async_copy(b_hbm.at[vidx.at[t1 + 2]], b1, sb1)

        return carry

    lax.fori_loop(0, NCHD // 2, body, 0)
    pltpu.make_async_copy(s0, _out_slice(NCHD - 2), ss0).wait()
    pltpu.make_async_copy(s1, _out_slice(NCHD - 1), ss1).wait()


_segsum_d1 = _segsum_factory(D1)
_segsum_d2 = _segsum_factory(D2)


def _pad_to(idx, n, fill):
    idx = idx.astype(jnp.int32)
    pad = jnp.full((n - E,), fill, jnp.int32)
    return jnp.concatenate([idx, pad])


def _seg_idx(graph):
    src = _pad_to(graph[0], ESEG, 0).reshape(NS, NCHS, CW)
    dst = _pad_to(graph[1], ESEG, N_NODE).reshape(NS, NCHS, CW)
    return src, dst


def _gcn_branch(x_pad, src, dst, deg, wg1, wg2, z64, z32):
    y1 = _matmul_split(x_pad, wg1)
    p = _segsum_d1(y1.reshape(2 * NPAD, D1), src, dst, z64)
    y2, dinv = _combine1(p, deg, wg2)
    q = _segsum_d2(y2.reshape(2 * NPAD, D2), src, dst, z32)
    return q, dinv


def kernel(pad_kmers_id_seq, enc_graph, dec_graph, drug_graph, dis_graph,
           drug_sim_feat, disease_sim_feat, Wp, bp, Wd, bd,
           Wg1_drug, Wg2_drug, Wg1_dis, Wg2_dis, Watt, batt, qatt,
           Wdec1, bdec1, Wdec2, bdec2):
    z64 = jnp.zeros((RPT, D1), jnp.float32)
    z32 = jnp.zeros((RPT, D2), jnp.float32)
    z16 = jnp.zeros((RPT, DDEG), jnp.float32)
    ones16 = jnp.ones((CW, DDEG), jnp.float32)
    xpad_drug = jnp.pad(drug_sim_feat, ((0, NPAD - N_NODE), (0, 0)))
    xpad_dis = jnp.pad(disease_sim_feat, ((0, NPAD - N_NODE), (0, 0)))

    src_drug, dst_drug = _seg_idx(drug_graph)
    src_dis, dst_dis = _seg_idx(dis_graph)
    deg2 = _deg_kernel(jnp.stack([dst_drug, dst_dis]), ones16, z16)
    q_drug, dinv_drug = _gcn_branch(xpad_drug, src_drug, dst_drug, deg2[0],
                                    Wg1_drug, Wg2_drug, z64, z32)
    q_dis, dinv_dis = _gcn_branch(xpad_dis, src_dis, dst_dis, deg2[1],
                                  Wg1_dis, Wg2_dis, z64, z32)

    bdec1_2d = bdec1.reshape(1, NHID2)
    zb = jnp.zeros((1, NHID2), jnp.float32)
    sim_drug, a_dec = _combine2(q_drug.reshape(2 * NPAD, D2), dinv_drug,
                                Wdec1[:NHID2], bdec1_2d)
    sim_dis, b_dec = _combine2(q_dis.reshape(2 * NPAD, D2), dinv_dis,
                               Wdec1[NHID2:], zb)

    u = _pad_to(dec_graph[0], EDEC, 0).reshape(NW, NCHD, CW)
    v = _pad_to(dec_graph[1], EDEC, 0).reshape(NW, NCHD, CW)
    ef = _edge_sum_kernel(a_dec, b_dec, u, v)
    pred = _decoder_mm(ef, Wdec2, bdec2.reshape(1, 2))[:E]

    drug_sim_out = sim_drug[:N_NODE]
    dis_sim_out = sim_dis[:N_NODE]
    return (pred, 0.0, 0.0, drug_sim_out, dis_sim_out)


# speedup vs baseline: 5.3605x; 1.0266x over previous
import functools

import jax
import jax.numpy as jnp
from jax import lax
from jax.experimental import pallas as pl
from jax.experimental.pallas import tpu as pltpu
from jax.experimental.pallas import tpu_sc as plsc

N_NODE = 10000
E = 320000
FDIM = 128
NHID2 = 64

NC = 2
NS = 16
NW = NC * NS
CW = 128
NPAD = 10240
RPT = NPAD // NS

NCHS = 158
ESEG = NS * NCHS * CW
D1 = 64
D2 = 32
DDEG = 16
NCHD = 80
EDEC = NW * NCHD * CW

_mesh = plsc.VectorSubcoreMesh(core_axis_name="c", subcore_axis_name="s")
_sc_params = pltpu.CompilerParams(use_tc_tiling_on_sc=False)


def _mm_split_body(x_ref, w_ref, o_ref):
    y = jnp.dot(x_ref[...], w_ref[...], preferred_element_type=jnp.float32)
    o_ref[0] = y[:, :NHID2]
    o_ref[1] = y[:, NHID2:]


def _matmul_split(x, w):
    return pl.pallas_call(
        _mm_split_body,
        out_shape=jax.ShapeDtypeStruct((2, NPAD, D1), jnp.float32),
    )(x, w)


def _comb1_body(p_ref, deg_ref, w_ref, y2_ref, dinv_ref):
    vals = jnp.concatenate([p_ref[:NPAD], p_ref[NPAD:]], axis=1)
    deg = deg_ref[:, 0:1]
    dinv = 1.0 / jnp.maximum(deg, 1.0)
    h = jnp.maximum(vals * dinv, 0.0)
    y2 = jnp.dot(h, w_ref[...], preferred_element_type=jnp.float32)
    y2_ref[0] = y2[:, :D2]
    y2_ref[1] = y2[:, D2:]
    dinv_ref[...] = jnp.broadcast_to(dinv, (NPAD, 16))


def _combine1(p, deg, w):
    return pl.pallas_call(
        _comb1_body,
        out_shape=(
            jax.ShapeDtypeStruct((2, NPAD, D2), jnp.float32),
            jax.ShapeDtypeStruct((NPAD, 16), jnp.float32),
        ),
    )(p, deg, w)


def _comb2_body(q_ref, dinv_ref, w_ref, b_ref, sim_ref, a_ref):
    sim = jnp.concatenate([q_ref[:NPAD], q_ref[NPAD:]], axis=1) * dinv_ref[:, 0:1]
    sim_ref[...] = sim
    a_ref[...] = jnp.dot(sim, w_ref[...], preferred_element_type=jnp.float32) + b_ref[...]


def _combine2(q, dinv, w, b2d):
    return pl.pallas_call(
        _comb2_body,
        out_shape=(
            jax.ShapeDtypeStruct((NPAD, NHID2), jnp.float32),
            jax.ShapeDtypeStruct((NPAD, NHID2), jnp.float32),
        ),
    )(q, dinv, w, b2d)


_DEC_BLK = 4096


def _dec_body(e_ref, w_ref, b_ref, o_ref):
    h = jnp.maximum(e_ref[...], 0.0)
    o_ref[...] = jnp.dot(h, w_ref[...], preferred_element_type=jnp.float32) + b_ref[...]


def _decoder_mm(ef, w, b2d):
    return pl.pallas_call(
        _dec_body,
        grid=(EDEC // _DEC_BLK,),
        in_specs=[
            pl.BlockSpec((_DEC_BLK, NHID2), lambda i: (i, 0)),
            pl.BlockSpec((NHID2, 2), lambda i: (0, 0)),
            pl.BlockSpec((1, 2), lambda i: (0, 0)),
        ],
        out_specs=pl.BlockSpec((_DEC_BLK, 2), lambda i: (i, 0)),
        out_shape=jax.ShapeDtypeStruct((EDEC, 2), jnp.float32),
    )(ef, w, b2d)


def _segsum_factory(D):
    @functools.partial(
        pl.kernel,
        mesh=_mesh,
        out_type=jax.ShapeDtypeStruct((NC * NPAD, D), jnp.float32),
        scratch_types=[
            pltpu.VMEM((NCHS, CW), jnp.int32),
            pltpu.VMEM((2, CW), jnp.int32),
            pltpu.VMEM((CW, D), jnp.float32),
            pltpu.VMEM((CW, D), jnp.float32),
            pltpu.VMEM_SHARED((NPAD, D), jnp.float32),
            pltpu.VMEM_SHARED((NPAD, D), jnp.float32),
            pltpu.SemaphoreType.DMA,
            pltpu.SemaphoreType.DMA,
            pltpu.SemaphoreType.DMA,
            pltpu.SemaphoreType.DMA,
            pltpu.SemaphoreType.DMA,
            pltpu.SemaphoreType.DMA,
        ],
        compiler_params=_sc_params,
    )
    def k(y_hbm, sidx_hbm, didx_hbm, z_hbm, out_hbm,
          sidx, dbuf, rows_a, rows_b, acc, ytab, ga, gb, sa, sb, d0, d1):
        c = lax.axis_index("c")
        s = lax.axis_index("s")
        pltpu.sync_copy(sidx_hbm.at[s], sidx)
        pltpu.sync_copy(y_hbm.at[pl.ds(c * NPAD + s * RPT, RPT)],
                        ytab.at[pl.ds(s * RPT, RPT)])
        pltpu.sync_copy(z_hbm, acc.at[pl.ds(s * RPT, RPT)])
        plsc.subcore_barrier()
        pltpu.async_copy(ytab.at[sidx.at[0]], rows_a, ga)
        pltpu.async_copy(ytab.at[sidx.at[1]], rows_b, gb)
        pltpu.async_copy(didx_hbm.at[s, 0], dbuf.at[0], d0)
        pltpu.async_copy(didx_hbm.at[s, 1], dbuf.at[1], d1)

        def body(i, carry):
            t0 = 2 * i
            t1 = t0 + 1
            pltpu.make_async_copy(ytab.at[sidx.at[t0]], rows_a, ga).wait()
            pltpu.make_async_copy(didx_hbm.at[s, t0], dbuf.at[0], d0).wait()
            pltpu.async_copy(rows_a, acc.at[dbuf.at[0]], sa, add=True)
            pltpu.make_async_copy(ytab.at[sidx.at[t1]], rows_b, gb).wait()
            pltpu.make_async_copy(didx_hbm.at[s, t1], dbuf.at[1], d1).wait()
            pltpu.async_copy(rows_b, acc.at[dbuf.at[1]], sb, add=True)
            pltpu.make_async_copy(rows_a, acc.at[dbuf.at[0]], sa).wait()

            @pl.when(t0 + 2 < NCHS)
            def _():
                pltpu.async_copy(ytab.at[sidx.at[t0 + 2]], rows_a, ga)
                pltpu.async_copy(didx_hbm.at[s, t0 + 2], dbuf.at[0], d0)

            pltpu.make_async_copy(rows_b, acc.at[dbuf.at[1]], sb).wait()

            @pl.when(t1 + 2 < NCHS)
            def _():
                pltpu.async_copy(ytab.at[sidx.at[t1 + 2]], rows_b, gb)
                pltpu.async_copy(didx_hbm.at[s, t1 + 2], dbuf.at[1], d1)

            return carry

        lax.fori_loop(0, NCHS // 2, body, 0)
        plsc.subcore_barrier()
        base = c * NPAD + s * RPT
        pltpu.sync_copy(acc.at[pl.ds(s * RPT, RPT)], out_hbm.at[pl.ds(base, RPT)])

    return k


@functools.partial(
    pl.kernel,
    mesh=_mesh,
    out_type=jax.ShapeDtypeStruct((NC, NPAD, DDEG), jnp.float32),
    scratch_types=[
        pltpu.VMEM((NCHS, CW), jnp.int32),
        pltpu.VMEM((CW, DDEG), jnp.float32),
        pltpu.VMEM_SHARED((NPAD, DDEG), jnp.float32),
        pltpu.SemaphoreType.DMA,
        pltpu.SemaphoreType.DMA,
    ],
    compiler_params=_sc_params,
)
def _deg_kernel(didx_hbm, ones_hbm, zdeg_hbm, out_hbm,
                didx, ones, dacc, da, db):
    c = lax.axis_index("c")
    s = lax.axis_index("s")
    pltpu.sync_copy(didx_hbm.at[c, s], didx)
    pltpu.sync_copy(ones_hbm, ones)
    pltpu.sync_copy(zdeg_hbm, dacc.at[pl.ds(s * RPT, RPT)])
    plsc.subcore_barrier()

    def body(i, carry):
        t0 = 2 * i
        t1 = t0 + 1
        pltpu.async_copy(ones, dacc.at[didx.at[t0]], da, add=True)
        pltpu.async_copy(ones, dacc.at[didx.at[t1]], db, add=True)
        pltpu.make_async_copy(ones, dacc.at[didx.at[t0]], da).wait()
        pltpu.make_async_copy(ones, dacc.at[didx.at[t1]], db).wait()
        return carry

    lax.fori_loop(0, NCHS // 2, body, 0)
    plsc.subcore_barrier()
    pltpu.sync_copy(dacc.at[pl.ds(s * RPT, RPT)],
                    out_hbm.at[c, pl.ds(s * RPT, RPT)])


@functools.partial(
    pl.kernel,
    mesh=_mesh,
    out_type=jax.ShapeDtypeStruct((EDEC, NHID2), jnp.float32),
    scratch_types=[
        pltpu.VMEM((NCHD, CW), jnp.int32),
        pltpu.VMEM((NCHD, CW), jnp.int32),
        pltpu.VMEM((CW, NHID2), jnp.float32),
        pltpu.VMEM((CW, NHID2), jnp.float32),
        pltpu.VMEM((CW, NHID2), jnp.float32),
        pltpu.VMEM((CW, NHID2), jnp.float32),
        pltpu.VMEM((CW, NHID2), jnp.float32),
        pltpu.VMEM((CW, NHID2), jnp.float32),
        pltpu.VMEM_SHARED((NPAD, NHID2), jnp.float32),
        pltpu.SemaphoreType.DMA,
        pltpu.SemaphoreType.DMA,
        pltpu.SemaphoreType.DMA,
        pltpu.SemaphoreType.DMA,
        pltpu.SemaphoreType.DMA,
        pltpu.SemaphoreType.DMA,
    ],
    compiler_params=_sc_params,
)
def _edge_sum_kernel(a_hbm, b_hbm, uidx_hbm, vidx_hbm, out_hbm,
                     uidx, vidx, a0, b0, a1, b1, s0, s1, atab,
                     sa0, sb0, sa1, sb1, ss0, ss1):
    c = lax.axis_index("c")
    s = lax.axis_index("s")
    wid = s * NC + c
    pltpu.sync_copy(uidx_hbm.at[wid], uidx)
    pltpu.sync_copy(vidx_hbm.at[wid], vidx)
    pltpu.sync_copy(a_hbm.at[pl.ds(s * RPT, RPT)], atab.at[pl.ds(s * RPT, RPT)])
    plsc.subcore_barrier()

    def _add(src_a, src_b, dst):
        def row(r, cc):
            for rr in range(4):
                for j in range(NHID2 // 16):
                    sl = pl.ds(j * 16, 16)
                    dst[r * 4 + rr, sl] = src_a[r * 4 + rr, sl] + src_b[r * 4 + rr, sl]
            return cc

        lax.fori_loop(0, CW // 4, row, 0)

    def _out_slice(t):
        return out_hbm.at[pl.ds((wid * NCHD + t) * CW, CW)]

    pltpu.async_copy(atab.at[uidx.at[0]], a0, sa0)
    pltpu.async_copy(b_hbm.at[vidx.at[0]], b0, sb0)
    pltpu.async_copy(atab.at[uidx.at[1]], a1, sa1)
    pltpu.async_copy(b_hbm.at[vidx.at[1]], b1, sb1)

    def body(i, carry):
        t0 = 2 * i
        t1 = t0 + 1
        pltpu.make_async_copy(atab.at[uidx.at[t0]], a0, sa0).wait()
        pltpu.make_async_copy(b_hbm.at[vidx.at[t0]], b0, sb0).wait()

        @pl.when(i > 0)
        def _():
            pltpu.make_async_copy(s0, _out_slice(t0 - 2), ss0).wait()

        _add(a0, b0, s0)
        pltpu.async_copy(s0, _out_slice(t0), ss0)

        @pl.when(t0 + 2 < NCHD)
        def _():
            pltpu.async_copy(atab.at[uidx.at[t0 + 2]], a0, sa0)
            pltpu.async_copy(b_hbm.at[vidx.at[t0 + 2]], b0, sb0)

        pltpu.make_async_copy(atab.at[uidx.at[t1]], a1, sa1).wait()
        pltpu.make_async_copy(b_hbm.at[vidx.at[t1]], b1, sb1).wait()

        @pl.when(i > 0)
        def _():
            pltpu.make_async_copy(s1, _out_slice(t1 - 2), ss1).wait()

        _add(a1, b1, s1)
        pltpu.async_copy(s1, _out_slice(t1), ss1)

        @pl.when(t1 + 2 < NCHD)
        def _():
            pltpu.async_copy(atab.at[uidx.at[t1 + 2]], a1, sa1)
            pltpu.async_copy(b_hbm.at[vidx.at[t1 + 2]], b1, sb1)

        return carry

    lax.fori_loop(0, NCHD // 2, body, 0)
    pltpu.make_async_copy(s0, _out_slice(NCHD - 2), ss0).wait()
    pltpu.make_async_copy(s1, _out_slice(NCHD - 1), ss1).wait()


_segsum_d1 = _segsum_factory(D1)
_segsum_d2 = _segsum_factory(D2)


def _pad_to(idx, n, fill):
    idx = idx.astype(jnp.int32)
    pad = jnp.full((n - E,), fill, jnp.int32)
    return jnp.concatenate([idx, pad])


def _seg_idx(graph):
    src = _pad_to(graph[0], ESEG, 0).reshape(NS, NCHS, CW)
    dst = _pad_to(graph[1], ESEG, N_NODE).reshape(NS, NCHS, CW)
    return src, dst


def _gcn_branch(x_pad, src, dst, deg, wg1, wg2, z64, z32):
    y1 = _matmul_split(x_pad, wg1)
    p = _segsum_d1(y1.reshape(2 * NPAD, D1), src, dst, z64)
    y2, dinv = _combine1(p, deg, wg2)
    q = _segsum_d2(y2.reshape(2 * NPAD, D2), src, dst, z32)
    return q, dinv


def kernel(pad_kmers_id_seq, enc_graph, dec_graph, drug_graph, dis_graph,
           drug_sim_feat, disease_sim_feat, Wp, bp, Wd, bd,
           Wg1_drug, Wg2_drug, Wg1_dis, Wg2_dis, Watt, batt, qatt,
           Wdec1, bdec1, Wdec2, bdec2):
    z64 = jnp.zeros((RPT, D1), jnp.float32)
    z32 = jnp.zeros((RPT, D2), jnp.float32)
    z16 = jnp.zeros((RPT, DDEG), jnp.float32)
    ones16 = jnp.ones((CW, DDEG), jnp.float32)
    xpad_drug = jnp.pad(drug_sim_feat, ((0, NPAD - N_NODE), (0, 0)))
    xpad_dis = jnp.pad(disease_sim_feat, ((0, NPAD - N_NODE), (0, 0)))

    src_drug, dst_drug = _seg_idx(drug_graph)
    src_dis, dst_dis = _seg_idx(dis_graph)
    deg2 = _deg_kernel(jnp.stack([dst_drug, dst_dis]), ones16, z16)
    q_drug, dinv_drug = _gcn_branch(xpad_drug, src_drug, dst_drug, deg2[0],
                                    Wg1_drug, Wg2_drug, z64, z32)
    q_dis, dinv_dis = _gcn_branch(xpad_dis, src_dis, dst_dis, deg2[1],
                                  Wg1_dis, Wg2_dis, z64, z32)

    bdec1_2d = bdec1.reshape(1, NHID2)
    zb = jnp.zeros((1, NHID2), jnp.float32)
    sim_drug, a_dec = _combine2(q_drug.reshape(2 * NPAD, D2), dinv_drug,
                                Wdec1[:NHID2], bdec1_2d)
    sim_dis, b_dec = _combine2(q_dis.reshape(2 * NPAD, D2), dinv_dis,
                               Wdec1[NHID2:], zb)

    u = _pad_to(dec_graph[0], EDEC, 0).reshape(NW, NCHD, CW)
    v = _pad_to(dec_graph[1], EDEC, 0).reshape(NW, NCHD, CW)
    ef = _edge_sum_kernel(a_dec, b_dec, u, v)
    pred = _decoder_mm(ef, Wdec2, bdec2.reshape(1, 2))[:E]

    drug_sim_out = sim_drug[:N_NODE]
    dis_sim_out = sim_dis[:N_NODE]
    return (pred, 0.0, 0.0, drug_sim_out, dis_sim_out)
